# Initial kernel scaffold; baseline (speedup 1.0000x reference)
import jax
import jax.numpy as jnp
from jax.experimental import pallas as pl


def kernel(output, target, segments):
    raise NotImplementedError("write your pallas kernel here")


# speedup vs baseline: 109.3095x; 109.3095x over previous
import functools

import jax
import jax.numpy as jnp
from jax import lax
from jax.experimental import pallas as pl
from jax.experimental.pallas import tpu as pltpu
from jax.experimental.pallas import tpu_sc as plsc

V, F, N = 4096, 64, 512
NPIX = N * N
NC, NS, L = 2, 16, 16
NW = NC * NS
ROWS_PER_TILE = V // NS
PIX_PER_TILE = NPIX // NW
GROUPS = ROWS_PER_TILE // L
PIX_STEPS = PIX_PER_TILE // L

_f32 = jnp.float32
_i32 = jnp.int32


def _sc_body(outm, seg, tgt, part,
             rows_v, cls_loc, cls_all, seg_v, tgt_v, hist_v, hist2_v,
             spmem_cls):
  cid = lax.axis_index("c")
  sid = lax.axis_index("s")
  wid = sid * NC + cid

  lane = jnp.arange(L, dtype=_i32)
  ones = jnp.ones((L,), _f32)
  zeros16 = jnp.zeros((L,), _f32)

  pltpu.sync_copy(outm.at[pl.ds(sid * ROWS_PER_TILE * F, ROWS_PER_TILE * F)],
                  rows_v)

  def group_body(g, _):
    row_base = (g * L + lane) * F
    m = jnp.full((L,), -jnp.inf, _f32)
    am = jnp.zeros((L,), _i32)
    for j in range(F):
      v = plsc.load_gather(rows_v, [row_base + j])
      upd = v > m
      am = jnp.where(upd, jnp.full((L,), j, _i32), am)
      m = jnp.where(upd, v, m)
    cls_loc[pl.ds(g * L, L)] = am
    return 0
  lax.fori_loop(0, GROUPS, group_body, 0)

  pltpu.sync_copy(cls_loc, spmem_cls.at[pl.ds(sid * ROWS_PER_TILE,
                                              ROWS_PER_TILE)])
  plsc.subcore_barrier()

  pltpu.sync_copy(spmem_cls, cls_all)
  base = wid * PIX_PER_TILE
  pltpu.sync_copy(seg.at[pl.ds(base, PIX_PER_TILE)], seg_v)
  pltpu.sync_copy(tgt.at[pl.ds(base, PIX_PER_TILE)], tgt_v)

  def zero_body(r, _):
    hist_v[pl.ds(r * L, L)] = zeros16
    return 0
  lax.fori_loop(0, 3 * F, zero_body, 0)

  def pix_body(i, _):
    s = seg_v[pl.ds(i * L, L)]
    t = tgt_v[pl.ds(i * L, L)]
    g = plsc.load_gather(cls_all, [s])
    t16 = t * L + lane
    plsc.addupdate_scatter(hist_v, [t16], ones)
    plsc.addupdate_scatter(hist_v, [(g + F) * L + lane], ones)
    plsc.addupdate_scatter(hist_v, [t16 + 2 * F * L], ones, mask=g == t)
    return 0
  lax.fori_loop(0, PIX_STEPS, pix_body, 0)

  def restage_body(r, _):
    hist2_v[r, :] = hist_v[pl.ds(r * L, L)]
    return 0
  lax.fori_loop(0, 3 * F, restage_body, 0)
  pltpu.sync_copy(hist2_v, part.at[cid, sid])


_sc_hist = functools.partial(
    pl.kernel,
    out_type=jax.ShapeDtypeStruct((NC, NS, 3 * F, L), _f32),
    mesh=plsc.VectorSubcoreMesh(core_axis_name="c", subcore_axis_name="s",
                                num_cores=NC, num_subcores=NS),
    compiler_params=pltpu.CompilerParams(needs_layout_passes=False),
    scratch_types=[
        pltpu.VMEM((ROWS_PER_TILE * F,), _f32),
        pltpu.VMEM((ROWS_PER_TILE,), _i32),
        pltpu.VMEM((V,), _i32),
        pltpu.VMEM((PIX_PER_TILE,), _i32),
        pltpu.VMEM((PIX_PER_TILE,), _i32),
        pltpu.VMEM((3 * F * L,), _f32),
        pltpu.VMEM((3 * F, L), _f32),
        pltpu.VMEM_SHARED((V,), _i32),
    ],
)(_sc_body)


def _dice_body(part_ref, out_ref):
  x = part_ref[...]
  s = jnp.sum(jnp.sum(x, axis=3), axis=0)
  t = s[0:1, :]
  o = s[1:2, :]
  inter = s[2:3, :]
  out_ref[...] = 2.0 * inter / (o + t + 1e-10)


_dice_reduce = pl.pallas_call(
    _dice_body,
    out_shape=jax.ShapeDtypeStruct((1, F), _f32),
)


@jax.jit
def kernel(output, target, segments):
  seg = segments.reshape(-1)
  tgt = target.reshape(-1)
  part = _sc_hist(output.reshape(-1), seg, tgt)
  part = part.reshape(NC * NS, 3, F, L)
  return _dice_reduce(part).reshape(F)


# speedup vs baseline: 113.1019x; 1.0347x over previous
import functools

import jax
import jax.numpy as jnp
from jax import lax
from jax.experimental import pallas as pl
from jax.experimental.pallas import tpu as pltpu
from jax.experimental.pallas import tpu_sc as plsc

V, F, N = 4096, 64, 512
NPIX = N * N
NC, NS, L = 2, 16, 16
NW = NC * NS
ROWS_PER_TILE = V // NS
PIX_PER_TILE = NPIX // NW
GROUPS = ROWS_PER_TILE // L
PIX_STEPS = PIX_PER_TILE // L

_f32 = jnp.float32
_i32 = jnp.int32


def _sc_body(outm, seg, tgt, part,
             rows_v, cls_loc, cls_all, seg_v, tgt_v, hist_v,
             spmem_cls, sem_s, sem_t):
  cid = lax.axis_index("c")
  sid = lax.axis_index("s")
  wid = sid * NC + cid

  lane = jnp.arange(L, dtype=_i32)
  ones = jnp.ones((L,), _f32)
  zeros16 = jnp.zeros((L,), _f32)

  base = wid * PIX_PER_TILE
  d_seg = pltpu.async_copy(seg.at[pl.ds(base, PIX_PER_TILE)], seg_v, sem_s)
  d_tgt = pltpu.async_copy(tgt.at[pl.ds(base, PIX_PER_TILE)], tgt_v, sem_t)

  pltpu.sync_copy(outm.at[pl.ds(sid * ROWS_PER_TILE * F, ROWS_PER_TILE * F)],
                  rows_v)

  def group_body(g, _):
    row_base = (g * L + lane) * F
    m = jnp.full((L,), -jnp.inf, _f32)
    am = jnp.zeros((L,), _i32)
    for j in range(F):
      v = plsc.load_gather(rows_v, [row_base + j])
      upd = v > m
      am = jnp.where(upd, jnp.full((L,), j, _i32), am)
      m = jnp.where(upd, v, m)
    cls_loc[pl.ds(g * L, L)] = am
    return 0
  lax.fori_loop(0, GROUPS, group_body, 0)

  pltpu.sync_copy(cls_loc, spmem_cls.at[pl.ds(sid * ROWS_PER_TILE,
                                              ROWS_PER_TILE)])
  plsc.subcore_barrier()

  pltpu.sync_copy(spmem_cls, cls_all)

  for r in range(3 * F):
    hist_v[pl.ds(r * L, L)] = zeros16

  d_seg.wait()
  d_tgt.wait()

  def pix_body(i, _):
    for u in range(4):
      off = (i * 4 + u) * L
      s = seg_v[pl.ds(off, L)]
      t = tgt_v[pl.ds(off, L)]
      g = plsc.load_gather(cls_all, [s])
      t16 = t * L + lane
      plsc.addupdate_scatter(hist_v, [t16], ones)
      plsc.addupdate_scatter(hist_v, [(g + F) * L + lane], ones)
      plsc.addupdate_scatter(hist_v, [t16 + 2 * F * L], ones, mask=g == t)
    return 0
  lax.fori_loop(0, PIX_STEPS // 4, pix_body, 0)

  pltpu.sync_copy(hist_v, part.at[cid, sid])


_sc_hist = functools.partial(
    pl.kernel,
    out_type=jax.ShapeDtypeStruct((NC, NS, 3 * F * L), _f32),
    mesh=plsc.VectorSubcoreMesh(core_axis_name="c", subcore_axis_name="s",
                                num_cores=NC, num_subcores=NS),
    compiler_params=pltpu.CompilerParams(needs_layout_passes=False),
    scratch_types=[
        pltpu.VMEM((ROWS_PER_TILE * F,), _f32),
        pltpu.VMEM((ROWS_PER_TILE,), _i32),
        pltpu.VMEM((V,), _i32),
        pltpu.VMEM((PIX_PER_TILE,), _i32),
        pltpu.VMEM((PIX_PER_TILE,), _i32),
        pltpu.VMEM((3 * F * L,), _f32),
        pltpu.VMEM_SHARED((V,), _i32),
        pltpu.SemaphoreType.DMA,
        pltpu.SemaphoreType.DMA,
    ],
)(_sc_body)


def _dice_body(part_ref, out_ref):
  x = part_ref[...]
  s = jnp.sum(jnp.sum(x, axis=3), axis=0)
  t = s[0:1, :]
  o = s[1:2, :]
  inter = s[2:3, :]
  out_ref[...] = 2.0 * inter / (o + t + 1e-10)


_dice_reduce = pl.pallas_call(
    _dice_body,
    out_shape=jax.ShapeDtypeStruct((1, F), _f32),
)


@jax.jit
def kernel(output, target, segments):
  seg = segments.reshape(-1)
  tgt = target.reshape(-1)
  part = _sc_hist(output.reshape(-1), seg, tgt)
  part = part.reshape(NC * NS, 3, F, L)
  return _dice_reduce(part).reshape(F)


# speedup vs baseline: 119.5446x; 1.0570x over previous
import functools

import jax
import jax.numpy as jnp
from jax import lax
from jax.experimental import pallas as pl
from jax.experimental.pallas import tpu as pltpu
from jax.experimental.pallas import tpu_sc as plsc

V, F, N = 4096, 64, 512
NC, NS, L = 2, 16, 16
NW = NC * NS
ROWS_PER_TILE = V // NS
PIXROWS_PER_TILE = N // NW
GROUPS = ROWS_PER_TILE // L
COLS = N // L

_f32 = jnp.float32
_i32 = jnp.int32


def _sc_body(outm, seg, tgt, part,
             rows_v, cls_loc, cls_all, seg_v, tgt_v, hist_v, hist2_v,
             spmem_cls, sem_s, sem_t):
  cid = lax.axis_index("c")
  sid = lax.axis_index("s")
  wid = sid * NC + cid

  lane = jnp.arange(L, dtype=_i32)
  ones = jnp.ones((L,), _f32)
  zeros16 = jnp.zeros((L,), _f32)

  rbase = wid * PIXROWS_PER_TILE
  d_seg = pltpu.async_copy(seg.at[pl.ds(rbase, PIXROWS_PER_TILE)], seg_v,
                           sem_s)
  d_tgt = pltpu.async_copy(tgt.at[pl.ds(rbase, PIXROWS_PER_TILE)], tgt_v,
                           sem_t)

  pltpu.sync_copy(outm.at[pl.ds(sid * ROWS_PER_TILE, ROWS_PER_TILE)], rows_v)

  def group_body(g, _):
    rows = g * L + lane
    m = jnp.full((L,), -jnp.inf, _f32)
    am = jnp.zeros((L,), _i32)
    for j in range(F):
      v = plsc.load_gather(rows_v, [rows, jnp.full((L,), j, _i32)])
      upd = v > m
      am = jnp.where(upd, jnp.full((L,), j, _i32), am)
      m = jnp.where(upd, v, m)
    cls_loc[pl.ds(g * L, L)] = am
    return 0
  lax.fori_loop(0, GROUPS, group_body, 0)

  pltpu.sync_copy(cls_loc, spmem_cls.at[pl.ds(sid * ROWS_PER_TILE,
                                              ROWS_PER_TILE)])
  plsc.subcore_barrier()

  pltpu.sync_copy(spmem_cls, cls_all)

  for r in range(3 * F):
    hist_v[pl.ds(r * L, L)] = zeros16

  d_seg.wait()
  d_tgt.wait()

  def pix_body(r, _):
    for c in range(COLS):
      s = seg_v[r, pl.ds(c * L, L)]
      t = tgt_v[r, pl.ds(c * L, L)]
      g = plsc.load_gather(cls_all, [s])
      t16 = t * L + lane
      plsc.addupdate_scatter(hist_v, [t16], ones)
      plsc.addupdate_scatter(hist_v, [(g + F) * L + lane], ones)
      plsc.addupdate_scatter(hist_v, [t16 + 2 * F * L], ones, mask=g == t)
    return 0
  lax.fori_loop(0, PIXROWS_PER_TILE, pix_body, 0)

  def restage_body(r, _):
    hist2_v[r, :] = hist_v[pl.ds(r * L, L)]
    return 0
  lax.fori_loop(0, 3 * F, restage_body, 0)
  pltpu.sync_copy(hist2_v, part.at[cid, sid])


_sc_hist = functools.partial(
    pl.kernel,
    out_type=jax.ShapeDtypeStruct((NC, NS, 3 * F, L), _f32),
    mesh=plsc.VectorSubcoreMesh(core_axis_name="c", subcore_axis_name="s",
                                num_cores=NC, num_subcores=NS),
    compiler_params=pltpu.CompilerParams(needs_layout_passes=False),
    scratch_types=[
        pltpu.VMEM((ROWS_PER_TILE, F), _f32),
        pltpu.VMEM((ROWS_PER_TILE,), _i32),
        pltpu.VMEM((V,), _i32),
        pltpu.VMEM((PIXROWS_PER_TILE, N), _i32),
        pltpu.VMEM((PIXROWS_PER_TILE, N), _i32),
        pltpu.VMEM((3 * F * L,), _f32),
        pltpu.VMEM((3 * F, L), _f32),
        pltpu.VMEM_SHARED((V,), _i32),
        pltpu.SemaphoreType.DMA,
        pltpu.SemaphoreType.DMA,
    ],
)(_sc_body)


def _dice_body(part_ref, out_ref):
  x = part_ref[...]
  s = jnp.sum(jnp.sum(jnp.sum(x, axis=3), axis=1), axis=0)
  t = s[0:F]
  o = s[F:2 * F]
  inter = s[2 * F:3 * F]
  out_ref[...] = 2.0 * inter / (o + t + 1e-10)


_dice_reduce = pl.pallas_call(
    _dice_body,
    out_shape=jax.ShapeDtypeStruct((F,), _f32),
)


@jax.jit
def kernel(output, target, segments):
  part = _sc_hist(output, segments, target)
  return _dice_reduce(part)
